# Initial kernel scaffold; baseline (speedup 1.0000x reference)
import jax
import jax.numpy as jnp
from jax.experimental import pallas as pl


def kernel(query_input, source_input, edge_index, Wq_w, Wq_b, Wk_w, Wk_b, Wv_w, Wv_b):
    raise NotImplementedError("write your pallas kernel here")


# speedup vs baseline: 27.6784x; 27.6784x over previous
import functools

import jax
import jax.numpy as jnp
from jax import lax
from jax.experimental import pallas as pl
from jax.experimental.pallas import tpu as pltpu
from jax.experimental.pallas import tpu_sc as plsc

C = 128
NSC = 2
NTILE = 16
NW = NSC * NTILE
BLK = 128
IDX_CH = 16
RB = 1024

_HIGH = jax.lax.Precision.HIGHEST


def _dot(a, b):
    return jax.lax.dot_general(a, b, (((1,), (0,)), ((), ())),
                               precision=_HIGH,
                               preferred_element_type=jnp.float32)


def _dot_t(a, b):
    return jax.lax.dot_general(a, b, (((0,), (0,)), ((), ())),
                               precision=_HIGH,
                               preferred_element_type=jnp.float32)


def _make_sc_degree(n_pad, blks_per_tile):
    rows_per_tile = n_pad // NTILE
    mesh = plsc.VectorSubcoreMesh(core_axis_name="c", subcore_axis_name="s")

    @functools.partial(
        pl.kernel, mesh=mesh,
        out_type=jax.ShapeDtypeStruct((NSC * n_pad,), jnp.float32),
        scratch_types=[
            pltpu.VMEM_SHARED((n_pad,), jnp.float32),
            pltpu.VMEM((blks_per_tile, BLK), jnp.int32),
            pltpu.VMEM((BLK,), jnp.float32),
            pltpu.VMEM((rows_per_tile,), jnp.float32),
        ],
    )
    def deg_kernel(col_hbm, deg_out, acc_sh, cidx, ones_v, vbuf):
        c = lax.axis_index("c")
        s = lax.axis_index("s")
        wid = c * NTILE + s

        def z16(i, _):
            vbuf[pl.ds(i * 16, 16)] = jnp.zeros((16,), jnp.float32)
            return 0
        lax.fori_loop(0, rows_per_tile // 16, z16, 0)
        for i in range(BLK // 16):
            ones_v[pl.ds(i * 16, 16)] = jnp.ones((16,), jnp.float32)
        pltpu.sync_copy(vbuf, acc_sh.at[pl.ds(s * rows_per_tile, rows_per_tile)])
        pltpu.sync_copy(col_hbm.at[pl.ds(wid * blks_per_tile, blks_per_tile)], cidx)
        plsc.subcore_barrier()

        def body(j, _):
            pltpu.sync_copy(ones_v, acc_sh.at[cidx.at[j]], add=True)
            return 0
        lax.fori_loop(0, blks_per_tile, body, 0)
        plsc.subcore_barrier()

        lo = s * rows_per_tile
        pltpu.sync_copy(acc_sh.at[pl.ds(lo, rows_per_tile)], vbuf)
        pltpu.sync_copy(vbuf, deg_out.at[pl.ds(c * n_pad + lo, rows_per_tile)])

    return deg_kernel


def _make_sc_gcn(n_pad, blks_per_tile):
    rows_per_tile = n_pad // NTILE
    mesh = plsc.VectorSubcoreMesh(core_axis_name="c", subcore_axis_name="s")

    @functools.partial(
        pl.kernel, mesh=mesh,
        out_type=jax.ShapeDtypeStruct((NSC * n_pad, C), jnp.float32),
        scratch_types=[
            pltpu.VMEM_SHARED((n_pad, C), jnp.float32),
            pltpu.VMEM((IDX_CH, BLK), jnp.int32),
            pltpu.VMEM((IDX_CH, BLK), jnp.int32),
            pltpu.VMEM((BLK, C), jnp.float32),
            pltpu.VMEM((BLK, C), jnp.float32),
            pltpu.SemaphoreType.DMA,
            pltpu.SemaphoreType.DMA,
        ],
    )
    def gcn_kernel(vsb_hbm, row_hbm, col_hbm, out_hbm,
                   acc_sh, ridx, cidx, rows0, rows1, sem0, sem1):
        c = lax.axis_index("c")
        s = lax.axis_index("s")
        wid = c * NTILE + s

        def zrow(r, _):
            for k in range(C // 16):
                rows0[r, pl.ds(k * 16, 16)] = jnp.zeros((16,), jnp.float32)
            return 0
        lax.fori_loop(0, BLK, zrow, 0)
        for t in range(rows_per_tile // BLK):
            pltpu.sync_copy(rows0, acc_sh.at[pl.ds(s * rows_per_tile + t * BLK, BLK)])
        plsc.subcore_barrier()

        def chunk(ch, _):
            base = wid * blks_per_tile + ch * IDX_CH
            pltpu.sync_copy(row_hbm.at[pl.ds(base, IDX_CH)], ridx)
            pltpu.sync_copy(col_hbm.at[pl.ds(base, IDX_CH)], cidx)
            pltpu.async_copy(vsb_hbm.at[ridx.at[0]], rows0, sem0)

            def pair(w, _):
                j0 = w * 2
                pltpu.make_async_copy(vsb_hbm.at[ridx.at[j0]], rows0, sem0).wait()
                pltpu.async_copy(vsb_hbm.at[ridx.at[j0 + 1]], rows1, sem1)
                pltpu.sync_copy(rows0, acc_sh.at[cidx.at[j0]], add=True)
                pltpu.make_async_copy(vsb_hbm.at[ridx.at[j0 + 1]], rows1, sem1).wait()

                @pl.when(j0 + 2 < IDX_CH)
                def _():
                    pltpu.async_copy(vsb_hbm.at[ridx.at[j0 + 2]], rows0, sem0)
                pltpu.sync_copy(rows1, acc_sh.at[cidx.at[j0 + 1]], add=True)
                return 0
            lax.fori_loop(0, IDX_CH // 2, pair, 0)
            return 0
        lax.fori_loop(0, blks_per_tile // IDX_CH, chunk, 0)
        plsc.subcore_barrier()

        for t in range(rows_per_tile // BLK):
            lo = s * rows_per_tile + t * BLK
            pltpu.sync_copy(acc_sh.at[pl.ds(lo, BLK)], rows0)
            pltpu.sync_copy(rows0, out_hbm.at[pl.ds(c * n_pad + lo, BLK)])

    return gcn_kernel


def _tc1_body(n, grid, q_ref, s_ref, wq_ref, bq_ref, wk_ref, bk_ref,
              wv_ref, bv_ref, deg_ref,
              qs_ref, vsb_ref, u_ref, stats_ref, u_acc, st_acc):
    i = pl.program_id(0)
    mask = (lax.broadcasted_iota(jnp.int32, (RB, 1), 0) + i * RB) < n

    q = _dot(q_ref[...], wq_ref[...]) + bq_ref[...]
    k = _dot(s_ref[...], wk_ref[...]) + bk_ref[...]
    v = _dot(s_ref[...], wv_ref[...]) + bv_ref[...]
    qm = jnp.where(mask, q, 0.0)
    km = jnp.where(mask, k, 0.0)
    vm = jnp.where(mask, v, 0.0)

    degb = deg_ref[0, :] + deg_ref[1, :]
    dinv = jnp.where(degb > 0, jax.lax.rsqrt(degb), 0.0)[:, None]

    qs_ref[...] = q
    vsb_ref[...] = jnp.where(mask, v * dinv, 0.0)

    st = jnp.concatenate([
        jnp.sum(km, axis=0, keepdims=True),
        jnp.sum(vm, axis=0, keepdims=True),
        jnp.full((1, C), jnp.sum(qm * qm), jnp.float32),
        jnp.full((1, C), jnp.sum(km * km), jnp.float32),
        jnp.zeros((4, C), jnp.float32),
    ], axis=0)

    @pl.when(i == 0)
    def _():
        u_acc[...] = jnp.zeros_like(u_acc)
        st_acc[...] = jnp.zeros_like(st_acc)

    u_acc[...] += _dot_t(km, vm)
    st_acc[...] += st

    @pl.when(i == grid - 1)
    def _():
        u_ref[...] = u_acc[...]
        stats_ref[...] = st_acc[...]


def _tc2_body(n, q_ref, deg_ref, u_ref, stats_ref, g0_ref, g1_ref, out_ref):
    qs = q_ref[...]
    sk = stats_ref[0, :]
    vsum = stats_ref[1, :]
    nq2 = stats_ref[2, 0]
    nk2 = stats_ref[3, 0]
    alpha = jax.lax.rsqrt(nq2) * jax.lax.rsqrt(nk2)

    num = _dot(qs, u_ref[...]) * alpha + vsum[None, :]
    den = _dot(qs, sk[:, None]) * alpha + jnp.float32(n)

    degb = deg_ref[0, :] + deg_ref[1, :]
    dinv = jnp.where(degb > 0, jax.lax.rsqrt(degb), 0.0)[:, None]
    out_ref[...] = num / den + dinv * (g0_ref[...] + g1_ref[...])


def kernel(query_input, source_input, edge_index,
           Wq_w, Wq_b, Wk_w, Wk_b, Wv_w, Wv_b):
    n = query_input.shape[0]
    e = edge_index.shape[1]

    n_pad = ((n + BLK) + BLK - 1) // BLK * BLK
    epb = NW * BLK * IDX_CH
    e_pad = (e + epb - 1) // epb * epb
    blks_per_tile = e_pad // (NW * BLK)
    pad = e_pad - e
    trash = n_pad - n

    ar = jnp.arange(pad, dtype=jnp.int32)
    row_p = jnp.concatenate([edge_index[0], ar % n])
    col_p = jnp.concatenate([edge_index[1], n + (ar % trash)])
    row2d = row_p.reshape(e_pad // BLK, BLK)
    col2d = col_p.reshape(e_pad // BLK, BLK)

    deg2 = _make_sc_degree(n_pad, blks_per_tile)(col2d).reshape(NSC, n_pad)

    grid1 = (n + RB - 1) // RB
    qs, vsb, u_mat, stats = pl.pallas_call(
        functools.partial(_tc1_body, n, grid1),
        grid=(grid1,),
        in_specs=[
            pl.BlockSpec((RB, C), lambda i: (i, 0)),
            pl.BlockSpec((RB, C), lambda i: (i, 0)),
            pl.BlockSpec((C, C), lambda i: (0, 0)),
            pl.BlockSpec((1, C), lambda i: (0, 0)),
            pl.BlockSpec((C, C), lambda i: (0, 0)),
            pl.BlockSpec((1, C), lambda i: (0, 0)),
            pl.BlockSpec((C, C), lambda i: (0, 0)),
            pl.BlockSpec((1, C), lambda i: (0, 0)),
            pl.BlockSpec((NSC, RB), lambda i: (0, i)),
        ],
        out_specs=[
            pl.BlockSpec((RB, C), lambda i: (i, 0)),
            pl.BlockSpec((RB, C), lambda i: (i, 0)),
            pl.BlockSpec((C, C), lambda i: (0, 0)),
            pl.BlockSpec((8, C), lambda i: (0, 0)),
        ],
        out_shape=[
            jax.ShapeDtypeStruct((n, C), jnp.float32),
            jax.ShapeDtypeStruct((n, C), jnp.float32),
            jax.ShapeDtypeStruct((C, C), jnp.float32),
            jax.ShapeDtypeStruct((8, C), jnp.float32),
        ],
        scratch_shapes=[
            pltpu.VMEM((C, C), jnp.float32),
            pltpu.VMEM((8, C), jnp.float32),
        ],
        compiler_params=pltpu.CompilerParams(
            dimension_semantics=("arbitrary",)),
    )(query_input, source_input, Wq_w, Wq_b.reshape(1, C), Wk_w,
      Wk_b.reshape(1, C), Wv_w, Wv_b.reshape(1, C), deg2)

    g = _make_sc_gcn(n_pad, blks_per_tile)(vsb, row2d, col2d)

    npb = n_pad // RB
    out = pl.pallas_call(
        functools.partial(_tc2_body, n),
        grid=(grid1,),
        in_specs=[
            pl.BlockSpec((RB, C), lambda i: (i, 0)),
            pl.BlockSpec((NSC, RB), lambda i: (0, i)),
            pl.BlockSpec((C, C), lambda i: (0, 0)),
            pl.BlockSpec((8, C), lambda i: (0, 0)),
            pl.BlockSpec((RB, C), lambda i: (i, 0)),
            pl.BlockSpec((RB, C), lambda i: (i + npb, 0)),
        ],
        out_specs=pl.BlockSpec((RB, C), lambda i: (i, 0)),
        out_shape=jax.ShapeDtypeStruct((n, C), jnp.float32),
    )(qs, deg2, u_mat, stats, g, g)

    return out


# speedup vs baseline: 30.9955x; 1.1198x over previous
import functools

import jax
import jax.numpy as jnp
from jax import lax
from jax.experimental import pallas as pl
from jax.experimental.pallas import tpu as pltpu
from jax.experimental.pallas import tpu_sc as plsc

C = 128
NSC = 2
NTILE = 16
NW = NSC * NTILE
BLK = 128
IDX_CH = 16
RB = 1024

def _dot(a, b):
    return jax.lax.dot_general(a, b, (((1,), (0,)), ((), ())),
                               preferred_element_type=jnp.float32)


def _dot_t(a, b):
    return jax.lax.dot_general(a, b, (((0,), (0,)), ((), ())),
                               preferred_element_type=jnp.float32)


def _make_sc_degree(n_pad, blks_per_tile):
    rows_per_tile = n_pad // NTILE
    mesh = plsc.VectorSubcoreMesh(core_axis_name="c", subcore_axis_name="s")

    @functools.partial(
        pl.kernel, mesh=mesh,
        out_type=jax.ShapeDtypeStruct((NSC * n_pad,), jnp.float32),
        scratch_types=[
            pltpu.VMEM_SHARED((n_pad,), jnp.float32),
            pltpu.VMEM((blks_per_tile, BLK), jnp.int32),
            pltpu.VMEM((BLK,), jnp.float32),
            pltpu.VMEM((rows_per_tile,), jnp.float32),
        ],
    )
    def deg_kernel(col_hbm, deg_out, acc_sh, cidx, ones_v, vbuf):
        c = lax.axis_index("c")
        s = lax.axis_index("s")
        wid = c * NTILE + s

        def z16(i, _):
            vbuf[pl.ds(i * 16, 16)] = jnp.zeros((16,), jnp.float32)
            return 0
        lax.fori_loop(0, rows_per_tile // 16, z16, 0)
        for i in range(BLK // 16):
            ones_v[pl.ds(i * 16, 16)] = jnp.ones((16,), jnp.float32)
        pltpu.sync_copy(vbuf, acc_sh.at[pl.ds(s * rows_per_tile, rows_per_tile)])
        pltpu.sync_copy(col_hbm.at[pl.ds(wid * blks_per_tile, blks_per_tile)], cidx)
        plsc.subcore_barrier()

        def body(j, _):
            pltpu.sync_copy(ones_v, acc_sh.at[cidx.at[j]], add=True)
            return 0
        lax.fori_loop(0, blks_per_tile, body, 0)
        plsc.subcore_barrier()

        lo = s * rows_per_tile
        pltpu.sync_copy(acc_sh.at[pl.ds(lo, rows_per_tile)], vbuf)
        pltpu.sync_copy(vbuf, deg_out.at[pl.ds(c * n_pad + lo, rows_per_tile)])

    return deg_kernel


def _make_sc_gcn(n_pad, blks_per_tile):
    rows_per_tile = n_pad // NTILE
    mesh = plsc.VectorSubcoreMesh(core_axis_name="c", subcore_axis_name="s")

    @functools.partial(
        pl.kernel, mesh=mesh,
        out_type=jax.ShapeDtypeStruct((NSC * n_pad, C), jnp.float32),
        scratch_types=[
            pltpu.VMEM_SHARED((n_pad, C), jnp.float32),
            pltpu.VMEM((IDX_CH, BLK), jnp.int32),
            pltpu.VMEM((IDX_CH, BLK), jnp.int32),
            pltpu.VMEM((BLK, C), jnp.float32),
            pltpu.VMEM((BLK, C), jnp.float32),
            pltpu.SemaphoreType.DMA,
            pltpu.SemaphoreType.DMA,
        ],
    )
    def gcn_kernel(vsb_hbm, row_hbm, col_hbm, out_hbm,
                   acc_sh, ridx, cidx, rows0, rows1, sem0, sem1):
        c = lax.axis_index("c")
        s = lax.axis_index("s")
        wid = c * NTILE + s

        def zrow(r, _):
            for k in range(C // 16):
                rows0[r, pl.ds(k * 16, 16)] = jnp.zeros((16,), jnp.float32)
            return 0
        lax.fori_loop(0, BLK, zrow, 0)
        for t in range(rows_per_tile // BLK):
            pltpu.sync_copy(rows0, acc_sh.at[pl.ds(s * rows_per_tile + t * BLK, BLK)])
        plsc.subcore_barrier()

        def chunk(ch, _):
            base = wid * blks_per_tile + ch * IDX_CH
            pltpu.sync_copy(row_hbm.at[pl.ds(base, IDX_CH)], ridx)
            pltpu.sync_copy(col_hbm.at[pl.ds(base, IDX_CH)], cidx)
            pltpu.async_copy(vsb_hbm.at[ridx.at[0]], rows0, sem0)

            def pair(w, _):
                j0 = w * 2
                pltpu.make_async_copy(vsb_hbm.at[ridx.at[j0]], rows0, sem0).wait()
                pltpu.async_copy(vsb_hbm.at[ridx.at[j0 + 1]], rows1, sem1)
                pltpu.sync_copy(rows0, acc_sh.at[cidx.at[j0]], add=True)
                pltpu.make_async_copy(vsb_hbm.at[ridx.at[j0 + 1]], rows1, sem1).wait()

                @pl.when(j0 + 2 < IDX_CH)
                def _():
                    pltpu.async_copy(vsb_hbm.at[ridx.at[j0 + 2]], rows0, sem0)
                pltpu.sync_copy(rows1, acc_sh.at[cidx.at[j0 + 1]], add=True)
                return 0
            lax.fori_loop(0, IDX_CH // 2, pair, 0)
            return 0
        lax.fori_loop(0, blks_per_tile // IDX_CH, chunk, 0)
        plsc.subcore_barrier()

        for t in range(rows_per_tile // BLK):
            lo = s * rows_per_tile + t * BLK
            pltpu.sync_copy(acc_sh.at[pl.ds(lo, BLK)], rows0)
            pltpu.sync_copy(rows0, out_hbm.at[pl.ds(c * n_pad + lo, BLK)])

    return gcn_kernel


def _tc1_body(n, grid, q_ref, s_ref, wq_ref, bq_ref, wk_ref, bk_ref,
              wv_ref, bv_ref,
              qs_ref, vs_ref, u_ref, stats_ref, u_acc, st_acc):
    i = pl.program_id(0)
    mask = (lax.broadcasted_iota(jnp.int32, (RB, 1), 0) + i * RB) < n

    q = _dot(q_ref[...], wq_ref[...]) + bq_ref[...]
    k = _dot(s_ref[...], wk_ref[...]) + bk_ref[...]
    v = _dot(s_ref[...], wv_ref[...]) + bv_ref[...]
    qm = jnp.where(mask, q, 0.0)
    km = jnp.where(mask, k, 0.0)
    vm = jnp.where(mask, v, 0.0)

    qs_ref[...] = q
    vs_ref[...] = v

    st = jnp.concatenate([
        jnp.sum(km, axis=0, keepdims=True),
        jnp.sum(vm, axis=0, keepdims=True),
        jnp.full((1, C), jnp.sum(qm * qm), jnp.float32),
        jnp.full((1, C), jnp.sum(km * km), jnp.float32),
        jnp.zeros((4, C), jnp.float32),
    ], axis=0)

    @pl.when(i == 0)
    def _():
        u_acc[...] = jnp.zeros_like(u_acc)
        st_acc[...] = jnp.zeros_like(st_acc)

    u_acc[...] += _dot_t(km, vm)
    st_acc[...] += st

    @pl.when(i == grid - 1)
    def _():
        u_ref[...] = u_acc[...]
        stats_ref[...] = st_acc[...]


def _tc_scale_body(vs_ref, deg_ref, vsb_ref):
    degb = deg_ref[0, :] + deg_ref[1, :]
    dinv = jnp.where(degb > 0, jax.lax.rsqrt(degb), 0.0)[:, None]
    vsb_ref[...] = vs_ref[...] * dinv


def _tc_attn_body(n, q_ref, u_ref, stats_ref, attn_ref):
    qs = q_ref[...]
    sk = stats_ref[0, :]
    vsum = stats_ref[1, :]
    nq2 = stats_ref[2, 0]
    nk2 = stats_ref[3, 0]
    alpha = jax.lax.rsqrt(nq2) * jax.lax.rsqrt(nk2)

    num = _dot(qs, u_ref[...]) * alpha + vsum[None, :]
    den = _dot(qs, sk[:, None]) * alpha + jnp.float32(n)
    attn_ref[...] = num / den


def _tc_comb_body(attn_ref, deg_ref, g0_ref, g1_ref, out_ref):
    degb = deg_ref[0, :] + deg_ref[1, :]
    dinv = jnp.where(degb > 0, jax.lax.rsqrt(degb), 0.0)[:, None]
    out_ref[...] = attn_ref[...] + dinv * (g0_ref[...] + g1_ref[...])


def kernel(query_input, source_input, edge_index,
           Wq_w, Wq_b, Wk_w, Wk_b, Wv_w, Wv_b):
    n = query_input.shape[0]
    e = edge_index.shape[1]

    n_pad = ((n + BLK) + BLK - 1) // BLK * BLK
    epb = NW * BLK * IDX_CH
    e_pad = (e + epb - 1) // epb * epb
    blks_per_tile = e_pad // (NW * BLK)
    pad = e_pad - e
    trash = n_pad - n

    ar = jnp.arange(pad, dtype=jnp.int32)
    row_p = jnp.concatenate([edge_index[0], ar % n])
    col_p = jnp.concatenate([edge_index[1], n + (ar % trash)])
    row2d = row_p.reshape(e_pad // BLK, BLK)
    col2d = col_p.reshape(e_pad // BLK, BLK)

    row_blk = pl.BlockSpec((RB, C), lambda i: (i, 0))
    full_cc = pl.BlockSpec((C, C), lambda i: (0, 0))
    full_1c = pl.BlockSpec((1, C), lambda i: (0, 0))
    full_st = pl.BlockSpec((8, C), lambda i: (0, 0))
    deg_blk = pl.BlockSpec((NSC, RB), lambda i: (0, i))
    grid1 = (n + RB - 1) // RB

    deg2 = _make_sc_degree(n_pad, blks_per_tile)(col2d).reshape(NSC, n_pad)

    qs, vs, u_mat, stats = pl.pallas_call(
        functools.partial(_tc1_body, n, grid1),
        grid=(grid1,),
        in_specs=[row_blk, row_blk, full_cc, full_1c, full_cc, full_1c,
                  full_cc, full_1c],
        out_specs=[row_blk, row_blk, full_cc, full_st],
        out_shape=[
            jax.ShapeDtypeStruct((n, C), jnp.float32),
            jax.ShapeDtypeStruct((n, C), jnp.float32),
            jax.ShapeDtypeStruct((C, C), jnp.float32),
            jax.ShapeDtypeStruct((8, C), jnp.float32),
        ],
        scratch_shapes=[
            pltpu.VMEM((C, C), jnp.float32),
            pltpu.VMEM((8, C), jnp.float32),
        ],
        compiler_params=pltpu.CompilerParams(
            dimension_semantics=("arbitrary",)),
    )(query_input, source_input, Wq_w, Wq_b.reshape(1, C), Wk_w,
      Wk_b.reshape(1, C), Wv_w, Wv_b.reshape(1, C))

    vsb = pl.pallas_call(
        _tc_scale_body,
        grid=(grid1,),
        in_specs=[row_blk, deg_blk],
        out_specs=row_blk,
        out_shape=jax.ShapeDtypeStruct((n, C), jnp.float32),
    )(vs, deg2)

    g = _make_sc_gcn(n_pad, blks_per_tile)(vsb, row2d, col2d)

    attn = pl.pallas_call(
        functools.partial(_tc_attn_body, n),
        grid=(grid1,),
        in_specs=[row_blk, full_cc, full_st],
        out_specs=row_blk,
        out_shape=jax.ShapeDtypeStruct((n, C), jnp.float32),
    )(qs, u_mat, stats)

    npb = n_pad // RB
    out = pl.pallas_call(
        _tc_comb_body,
        grid=(grid1,),
        in_specs=[row_blk, deg_blk, row_blk,
                  pl.BlockSpec((RB, C), lambda i: (i + npb, 0))],
        out_specs=row_blk,
        out_shape=jax.ShapeDtypeStruct((n, C), jnp.float32),
    )(attn, deg2, g, g)

    return out
